# Initial kernel scaffold; baseline (speedup 1.0000x reference)
import jax
import jax.numpy as jnp
from jax.experimental import pallas as pl


def kernel(values, segment_heads):
    raise NotImplementedError("write your pallas kernel here")


# speedup vs baseline: 10.0989x; 10.0989x over previous
import jax
import jax.numpy as jnp
from jax.experimental import pallas as pl
from jax.experimental.pallas import tpu as pltpu

_N = 6_400_000
_LANES = 128
_ROWS = _N // _LANES
_R = 1024
_GRID = (_ROWS + _R - 1) // _R


def _seg_scan(s, f, axis, dists):
    for d in dists:
        if axis == 1:
            zs = jnp.zeros(s.shape[:1] + (d,), s.dtype)
            s_sh = jnp.concatenate([zs, s[:, :-d]], axis=1)
            f_sh = jnp.concatenate([zs, f[:, :-d]], axis=1)
        else:
            zs = jnp.zeros((d,) + s.shape[1:], s.dtype)
            s_sh = jnp.concatenate([zs, s[:-d, :]], axis=0)
            f_sh = jnp.concatenate([zs, f[:-d, :]], axis=0)
        s = s + (1.0 - f) * s_sh
        f = jnp.maximum(f, f_sh)
    return s, f


def _body(v_ref, h_ref, o_ref, carry_sm):
    pid = pl.program_id(0)

    v = v_ref[...]
    f0 = h_ref[...].astype(jnp.float32)

    s, f = _seg_scan(v, f0, axis=1, dists=(1, 2, 4, 8, 16, 32, 64))

    rs = jnp.reshape(s[:, _LANES - 1 : _LANES], (8, 128))
    rf = jnp.reshape(f[:, _LANES - 1 : _LANES], (8, 128))
    rs, rf = _seg_scan(rs, rf, axis=1, dists=(1, 2, 4, 8, 16, 32, 64))
    for d in (1, 2, 4):
        zs = jnp.zeros((d, 128), jnp.float32)
        es = jnp.concatenate([zs, jnp.broadcast_to(rs[:, 127:128], (8, 128))[:-d]], 0)
        ef = jnp.concatenate([zs, jnp.broadcast_to(rf[:, 127:128], (8, 128))[:-d]], 0)
        rs = rs + (1.0 - rf) * es
        rf = jnp.maximum(rf, ef)

    flat_s = jnp.reshape(rs, (1024, 1))
    flat_f = jnp.reshape(rf, (1024, 1))
    ex_s = jnp.concatenate([jnp.zeros((1, 1), jnp.float32), flat_s[:-1]], 0)
    ex_f = jnp.concatenate([jnp.zeros((1, 1), jnp.float32), flat_f[:-1]], 0)

    carry = jnp.where(pid == 0, 0.0, carry_sm[0])
    crow = ex_s + (1.0 - ex_f) * carry
    out_incl = s + (1.0 - f) * crow
    o_ref[...] = out_incl - v

    blk_s = jnp.sum(flat_s[1023:1024, 0:1])
    blk_f = jnp.sum(flat_f[1023:1024, 0:1])
    carry_sm[0] = blk_s + (1.0 - blk_f) * carry


def kernel(values, segment_heads):
    v2 = values.reshape(_ROWS, _LANES)
    h2 = segment_heads.reshape(_ROWS, _LANES)
    out = pl.pallas_call(
        _body,
        grid=(_GRID,),
        in_specs=[
            pl.BlockSpec((_R, _LANES), lambda i: (i, 0)),
            pl.BlockSpec((_R, _LANES), lambda i: (i, 0)),
        ],
        out_specs=pl.BlockSpec((_R, _LANES), lambda i: (i, 0)),
        out_shape=jax.ShapeDtypeStruct((_ROWS, _LANES), jnp.float32),
        scratch_shapes=[pltpu.SMEM((1,), jnp.float32)],
    )(v2, h2)
    return out.reshape(_N)


# speedup vs baseline: 11.0309x; 1.0923x over previous
import functools

import jax
import jax.numpy as jnp
from jax import lax
from jax.experimental import pallas as pl
from jax.experimental.pallas import tpu as pltpu
from jax.experimental.pallas import tpu_sc as plsc

_N = 6_400_000
_NW = 32
_CHUNK = _N // _NW
_B = 10_000
_NB = _CHUNK // _B
_U = 5
_VPB = _B // 16
_IOTA = None

_LANE15 = 15

_GATHER_DNUMS = lax.GatherDimensionNumbers(
    offset_dims=(), collapsed_slice_dims=(0,), start_index_map=(0,)
)


def _take16(x, idx):
    return lax.gather(
        x,
        idx[:, None],
        _GATHER_DNUMS,
        slice_sizes=(1,),
        mode=lax.GatherScatterMode.PROMISE_IN_BOUNDS,
    )


def _seg_scan_vreg(v, h, carry_vec, iota16, lane15_idx):
    cs = plsc.cumsum(v)
    cse = cs - v
    hm = h > 0
    hidx = jnp.where(hm, iota16, -1)
    start = plsc.cummax(hidx)
    offv = _take16(cse, jnp.maximum(start, 0))
    negm = start < 0
    w0 = jnp.where(negm, cse, cse - offv)
    negf = jnp.where(negm, 1.0, 0.0)
    out = w0 + negf * carry_vec

    start_b = _take16(start, lane15_idx)
    tot_b = _take16(cs, lane15_idx)
    off_b = _take16(offv, lane15_idx)
    no_head = start_b < 0
    alpha = jnp.where(no_head, 1.0, 0.0)
    beta = jnp.where(no_head, tot_b, tot_b - off_b)
    return out, hm, alpha, beta


def _k1_body(v_hbm, h_hbm, o_hbm, agga_hbm, aggp_hbm, vbuf, hbuf, obuf, abuf, pbuf):
    wid = lax.axis_index("c") * 16 + lax.axis_index("s")
    base = wid * _CHUNK
    iota16 = lax.iota(jnp.int32, 16)
    lane15_idx = iota16 * 0 + _LANE15

    def block_body(b, st):
        carry, pvec = st
        off = base + b * _B
        pltpu.sync_copy(v_hbm.at[pl.ds(off, _B)], vbuf)
        pltpu.sync_copy(h_hbm.at[pl.ds(off, _B)], hbuf)

        def vreg_body(k, st2):
            carry, pvec = st2
            for u in range(_U):
                i = k * _U + u
                v = vbuf[pl.ds(i * 16, 16)]
                h = hbuf[pl.ds(i * 16, 16)]
                out, hm, alpha, beta = _seg_scan_vreg(v, h, carry, iota16, lane15_idx)
                obuf[pl.ds(i * 16, 16)] = out
                carry = alpha * carry + beta
                hpos = jnp.where(hm, iota16 + (b * _B + i * 16), _CHUNK)
                pvec = jnp.minimum(pvec, hpos)
            return carry, pvec

        carry, pvec = lax.fori_loop(0, _VPB // _U, vreg_body, (carry, pvec))
        pltpu.sync_copy(obuf, o_hbm.at[pl.ds(off, _B)])
        return carry, pvec

    carry0 = jnp.zeros((16,), jnp.float32)
    pvec0 = jnp.full((16,), _CHUNK, jnp.int32)
    carry, pvec = lax.fori_loop(0, _NB, block_body, (carry0, pvec0))

    pmin = jnp.min(pvec)
    abuf[...] = carry
    pbuf[...] = iota16 * 0 + pmin
    pltpu.sync_copy(abuf, agga_hbm.at[pl.ds(wid * 16, 16)])
    pltpu.sync_copy(pbuf, aggp_hbm.at[pl.ds(wid * 16, 16)])


def _k2_body(o1_hbm, agga_hbm, aggp_hbm, o2_hbm, buf, abuf, pbuf):
    wid = lax.axis_index("c") * 16 + lax.axis_index("s")
    base = wid * _CHUNK
    iota16 = lax.iota(jnp.int32, 16)

    pltpu.sync_copy(agga_hbm, abuf)
    pltpu.sync_copy(aggp_hbm, pbuf)

    gidx = iota16 * 16
    a_lo = plsc.load_gather(abuf, [gidx])
    a_hi = plsc.load_gather(abuf, [gidx + 256])
    p_lo = plsc.load_gather(pbuf, [gidx])
    p_hi = plsc.load_gather(pbuf, [gidx + 256])

    def incl_scan(a, f, carry_in):
        cs = plsc.cumsum(a)
        hidx = jnp.where(f, iota16, -1)
        start = plsc.cummax(hidx)
        offv = _take16(cs - a, jnp.maximum(start, 0))
        return jnp.where(start < 0, cs + carry_in, cs - offv)

    lane15 = iota16 * 0 + _LANE15
    incl_lo = incl_scan(a_lo, p_lo < _CHUNK, jnp.zeros((16,), jnp.float32))
    c16 = _take16(incl_lo, lane15)
    incl_hi = incl_scan(a_hi, p_hi < _CHUNK, c16)

    jm1 = jnp.maximum(wid - 1, 0)
    jlo = jnp.minimum(jm1, 15)
    jhi = jnp.maximum(jnp.minimum(wid - 17, 15), 0)
    t_lo = _take16(incl_lo, jnp.broadcast_to(jlo, (16,)))
    t_hi = _take16(incl_hi, jnp.broadcast_to(jhi, (16,)))
    use_lo = jnp.where(wid <= 16, 1.0, 0.0)
    cvec = (t_lo * use_lo + t_hi * (1.0 - use_lo)) * jnp.where(wid == 0, 0.0, 1.0)

    pvec = pbuf[pl.ds(wid * 16, 16)]
    p_scalar = jnp.max(pvec)

    def block_body(b, _):
        off = base + b * _B
        pltpu.sync_copy(o1_hbm.at[pl.ds(off, _B)], buf)

        @pl.when(b * _B < p_scalar)
        def _():
            def vreg_body(i, _):
                g = iota16 + (b * _B + i * 16)
                x = buf[pl.ds(i * 16, 16)]
                buf[pl.ds(i * 16, 16)] = x + jnp.where(g < pvec, cvec, 0.0)
                return 0
            lax.fori_loop(0, _VPB, vreg_body, 0)

        pltpu.sync_copy(buf, o2_hbm.at[pl.ds(off, _B)])
        return 0

    lax.fori_loop(0, _NB, block_body, 0)


def kernel(values, segment_heads):
    heads_i32 = segment_heads.astype(jnp.int32)
    mesh = plsc.VectorSubcoreMesh(core_axis_name="c", subcore_axis_name="s")
    params = pltpu.CompilerParams(needs_layout_passes=False)

    k1 = pl.kernel(
        _k1_body,
        out_type=(
            jax.ShapeDtypeStruct((_N,), jnp.float32),
            jax.ShapeDtypeStruct((_NW * 16,), jnp.float32),
            jax.ShapeDtypeStruct((_NW * 16,), jnp.int32),
        ),
        mesh=mesh,
        compiler_params=params,
        scratch_types=[
            pltpu.VMEM((_B,), jnp.float32),
            pltpu.VMEM((_B,), jnp.int32),
            pltpu.VMEM((_B,), jnp.float32),
            pltpu.VMEM((16,), jnp.float32),
            pltpu.VMEM((16,), jnp.int32),
        ],
    )
    o1, agga, aggp = k1(values, heads_i32)

    k2 = pl.kernel(
        _k2_body,
        out_type=jax.ShapeDtypeStruct((_N,), jnp.float32),
        mesh=mesh,
        compiler_params=params,
        scratch_types=[
            pltpu.VMEM((_B,), jnp.float32),
            pltpu.VMEM((_NW * 16,), jnp.float32),
            pltpu.VMEM((_NW * 16,), jnp.int32),
        ],
    )
    return k2(o1, agga, aggp)
